# baseline (speedup 1.0000x reference)
import functools

import jax
import jax.numpy as jnp
from jax import lax
from jax.experimental import pallas as pl
from jax.experimental.pallas import tpu as pltpu
from jax.experimental.pallas import tpu_sc as plsc

TOTAL_TOKENS = 16384
EMB = 512
_NC = 2
_NS = 16
_NW = _NC * _NS
_BPW = TOTAL_TOKENS // _NW
_CH = 64
_NCHUNK = _BPW // _CH
_NBUF = 3
_INFLIGHT = 2


def _make_gather():
    mesh = plsc.VectorSubcoreMesh(core_axis_name="c", subcore_axis_name="s")

    @functools.partial(
        pl.kernel,
        mesh=mesh,
        out_type=jax.ShapeDtypeStruct((TOTAL_TOKENS, EMB), jnp.float32),
        scratch_types=[
            pltpu.VMEM((_BPW,), jnp.int32),
        ] + [pltpu.VMEM((_CH, EMB), jnp.float32)] * _NBUF
          + [pltpu.SemaphoreType.DMA] * (2 * _NBUF),
    )
    def gather_k(idx_hbm, table_hbm, out_hbm, idx_v, *bufs_and_sems):
        bufs = bufs_and_sems[:_NBUF]
        gsems = bufs_and_sems[_NBUF:2 * _NBUF]
        wsems = bufs_and_sems[2 * _NBUF:]
        wid = lax.axis_index("s") * _NC + lax.axis_index("c")
        base = wid * _BPW
        pltpu.sync_copy(idx_hbm.at[pl.ds(base, _BPW)], idx_v)

        def start_gather(c):
            slot = c % _NBUF
            return pltpu.async_copy(
                table_hbm.at[idx_v.at[pl.ds(c * _CH, _CH)]],
                bufs[slot], gsems[slot])

        def start_write(c):
            slot = c % _NBUF
            return pltpu.async_copy(
                bufs[slot], out_hbm.at[pl.ds(base + c * _CH, _CH)],
                wsems[slot])

        pend_g = {c: start_gather(c) for c in range(_INFLIGHT)}
        pend_w = {}
        for c in range(_NCHUNK):
            pend_g.pop(c).wait()
            pend_w[c] = start_write(c)
            nc = c + _INFLIGHT
            if nc < _NCHUNK:
                prev = nc - _NBUF
                if prev >= 0:
                    pend_w.pop(prev).wait()
                pend_g[nc] = start_gather(nc)
        for w in pend_w.values():
            w.wait()

    return gather_k


_gather = _make_gather()


def kernel(indices, cu_seqlens, table):
    del cu_seqlens
    return _gather(indices.astype(jnp.int32), table)


# speedup vs baseline: 1.0378x; 1.0378x over previous
import functools

import jax
import jax.numpy as jnp
from jax import lax
from jax.experimental import pallas as pl
from jax.experimental.pallas import tpu as pltpu
from jax.experimental.pallas import tpu_sc as plsc

TOTAL_TOKENS = 16384
EMB = 512
_NC = 2
_NS = 16
_NW = _NC * _NS
_BPW = TOTAL_TOKENS // _NW
_CH = 16
_NCHUNK = _BPW // _CH
_NBUF = 14
_INFLIGHT = 12


def _make_gather():
    mesh = plsc.VectorSubcoreMesh(core_axis_name="c", subcore_axis_name="s")

    @functools.partial(
        pl.kernel,
        mesh=mesh,
        out_type=jax.ShapeDtypeStruct((TOTAL_TOKENS, EMB), jnp.float32),
        scratch_types=[
            pltpu.VMEM((_BPW,), jnp.int32),
        ] + [pltpu.VMEM((_CH, EMB), jnp.float32)] * _NBUF
          + [pltpu.SemaphoreType.DMA] * (2 * _NBUF),
    )
    def gather_k(idx_hbm, table_hbm, out_hbm, idx_v, *bufs_and_sems):
        bufs = bufs_and_sems[:_NBUF]
        gsems = bufs_and_sems[_NBUF:2 * _NBUF]
        wsems = bufs_and_sems[2 * _NBUF:]
        wid = lax.axis_index("s") * _NC + lax.axis_index("c")
        base = wid * _BPW
        pltpu.sync_copy(idx_hbm.at[pl.ds(base, _BPW)], idx_v)

        def start_gather(c):
            slot = c % _NBUF
            return pltpu.async_copy(
                table_hbm.at[idx_v.at[pl.ds(c * _CH, _CH)]],
                bufs[slot], gsems[slot])

        def start_write(c):
            slot = c % _NBUF
            return pltpu.async_copy(
                bufs[slot], out_hbm.at[pl.ds(base + c * _CH, _CH)],
                wsems[slot])

        pend_g = {c: start_gather(c) for c in range(_INFLIGHT)}
        pend_w = {}
        for c in range(_NCHUNK):
            pend_g.pop(c).wait()
            pend_w[c] = start_write(c)
            nc = c + _INFLIGHT
            if nc < _NCHUNK:
                prev = nc - _NBUF
                if prev >= 0:
                    pend_w.pop(prev).wait()
                pend_g[nc] = start_gather(nc)
        for w in pend_w.values():
            w.wait()

    return gather_k


_gather = _make_gather()


def kernel(indices, cu_seqlens, table):
    del cu_seqlens
    return _gather(indices.astype(jnp.int32), table)


# speedup vs baseline: 1.0448x; 1.0067x over previous
import functools

import jax
import jax.numpy as jnp
from jax import lax
from jax.experimental import pallas as pl
from jax.experimental.pallas import tpu as pltpu
from jax.experimental.pallas import tpu_sc as plsc

TOTAL_TOKENS = 16384
EMB = 512
_NC = 2
_NS = 16
_NW = _NC * _NS
_BPW = TOTAL_TOKENS // _NW
_CH = 32
_NCHUNK = _BPW // _CH
_NBUF = 7
_INFLIGHT = 6


def _make_gather():
    mesh = plsc.VectorSubcoreMesh(core_axis_name="c", subcore_axis_name="s")

    @functools.partial(
        pl.kernel,
        mesh=mesh,
        out_type=jax.ShapeDtypeStruct((TOTAL_TOKENS, EMB), jnp.float32),
        scratch_types=[
            pltpu.VMEM((_BPW,), jnp.int32),
        ] + [pltpu.VMEM((_CH, EMB), jnp.float32)] * _NBUF
          + [pltpu.SemaphoreType.DMA] * (2 * _NBUF),
    )
    def gather_k(idx_hbm, table_hbm, out_hbm, idx_v, *bufs_and_sems):
        bufs = bufs_and_sems[:_NBUF]
        gsems = bufs_and_sems[_NBUF:2 * _NBUF]
        wsems = bufs_and_sems[2 * _NBUF:]
        wid = lax.axis_index("s") * _NC + lax.axis_index("c")
        base = wid * _BPW
        pltpu.sync_copy(idx_hbm.at[pl.ds(base, _BPW)], idx_v)

        def start_gather(c):
            slot = c % _NBUF
            return pltpu.async_copy(
                table_hbm.at[idx_v.at[pl.ds(c * _CH, _CH)]],
                bufs[slot], gsems[slot])

        def start_write(c):
            slot = c % _NBUF
            return pltpu.async_copy(
                bufs[slot], out_hbm.at[pl.ds(base + c * _CH, _CH)],
                wsems[slot])

        pend_g = {c: start_gather(c) for c in range(_INFLIGHT)}
        pend_w = {}
        for c in range(_NCHUNK):
            pend_g.pop(c).wait()
            pend_w[c] = start_write(c)
            nc = c + _INFLIGHT
            if nc < _NCHUNK:
                prev = nc - _NBUF
                if prev >= 0:
                    pend_w.pop(prev).wait()
                pend_g[nc] = start_gather(nc)
        for w in pend_w.values():
            w.wait()

    return gather_k


_gather = _make_gather()


def kernel(indices, cu_seqlens, table):
    del cu_seqlens
    return _gather(indices.astype(jnp.int32), table)
